# baseline (speedup 1.0000x reference)
import functools

import jax
import jax.numpy as jnp
from jax import lax
from jax.experimental import pallas as pl
from jax.experimental.pallas import tpu as pltpu
from jax.experimental.pallas import tpu_sc as plsc

D = 128
N_NODES = 10000
NPAD = 10240
RPT = NPAD // 16
NC, NS = 2, 16
NW = NC * NS
N_EDGES_IN = 320000
E_PAD = 327680
EPW = E_PAD // NW
CK = 128
NCH = EPW // CK
NT = 16384
TPW = NT // NW
TCK = 128
NTCH = TPW // TCK
MARGIN = 0.1
VAE_BETA = 0.2
MAX_NORM = 1.5
BN = 1000
NBLK = N_NODES // BN


def _col0(x):
    return lax.broadcasted_iota(jnp.int32, x.shape, 1) == 0


def _logmap0(x):
    t = jnp.clip(x[:, :1], 1.0 + 1e-7, None)
    dist = jnp.log(t + jnp.sqrt(t * t - 1.0))
    n2 = jnp.maximum(jnp.sum(x * x, axis=-1, keepdims=True) - x[:, :1] * x[:, :1], 0.0)
    n = jnp.clip(jnp.sqrt(n2), 1e-7, None)
    return jnp.where(_col0(x), 0.0, x * (dist / n))


def _expmap0(u, project):
    n2 = jnp.maximum(jnp.sum(u * u, axis=-1, keepdims=True) - u[:, :1] * u[:, :1], 0.0)
    n = jnp.clip(jnp.sqrt(n2), 1e-7, None)
    nc = jnp.minimum(n, MAX_NORM) if project else n
    e = jnp.exp(nc)
    ei = 1.0 / e
    cosh = 0.5 * (e + ei)
    sinh = 0.5 * (e - ei)
    return jnp.where(_col0(u), cosh, u * (sinh / n))


def _sqdist(x, y):
    ip = jnp.sum(x * y, axis=-1) - 2.0 * x[:, 0] * y[:, 0]
    return jnp.maximum(-2.0 - 2.0 * ip, 0.0)


def _moe_block(x, gW, gb, eWt, eb):
    g = jnp.dot(x, gW, preferred_element_type=jnp.float32) + gb
    g = g - jnp.max(g, axis=-1, keepdims=True)
    w = jnp.exp(g)
    w = w / jnp.sum(w, axis=-1, keepdims=True)
    acc = jnp.zeros((x.shape[0], D), jnp.float32)
    for ex in range(8):
        ye = jnp.dot(x, eWt[:, ex * D:(ex + 1) * D], preferred_element_type=jnp.float32)
        acc = acc + w[:, ex:ex + 1] * (ye + eb[ex:ex + 1, :])
    return acc


def _dense_body(emb_ref, eps_ref, tr_gW, tr_gb, tr_eWt, tr_eb,
                mu_W, mu_b, lv_W, lv_b, fc1_W, fc1_b, fc2_W, fc2_b,
                dec_gW, dec_gb, dec_eWt, dec_eb,
                mui_ref, recon_ref, kl_ref):
    x = emb_ref[...]
    t = _logmap0(x)
    m = _moe_block(t, tr_gW[...], tr_gb[...], tr_eWt[...], tr_eb[...])
    mui_ref[...] = m
    mu = jnp.dot(m, mu_W[...], preferred_element_type=jnp.float32) + mu_b[...]
    lv = jnp.dot(m, lv_W[...], preferred_element_type=jnp.float32) + lv_b[...]
    std = jnp.exp(0.5 * lv)
    z = _expmap0(mu + eps_ref[...] * std, True)
    zt = _logmap0(z)
    h = jnp.maximum(jnp.dot(zt, fc1_W[...], preferred_element_type=jnp.float32) + fc1_b[...], 0.0)
    h = jnp.dot(h, fc2_W[...], preferred_element_type=jnp.float32) + fc2_b[...]
    h = _moe_block(h, dec_gW[...], dec_gb[...], dec_eWt[...], dec_eb[...])
    recon = _expmap0(h, True)
    target = _expmap0(m, False)
    rblk = jnp.sum(_sqdist(recon, target))
    kblk = jnp.sum(1.0 + lv - mu * mu - jnp.exp(lv))

    @pl.when(pl.program_id(0) == 0)
    def _init():
        recon_ref[...] = jnp.zeros_like(recon_ref)
        kl_ref[...] = jnp.zeros_like(kl_ref)

    recon_ref[...] = recon_ref[...] + jnp.full((1, 1), 0.0, jnp.float32) + rblk
    kl_ref[...] = kl_ref[...] + kblk


def _full_spec(shape):
    return pl.BlockSpec(shape, lambda i: tuple(0 for _ in shape))


_dense_call = pl.pallas_call(
    _dense_body,
    grid=(NBLK,),
    in_specs=[
        pl.BlockSpec((BN, D), lambda i: (i, 0)),
        pl.BlockSpec((BN, D), lambda i: (i, 0)),
        _full_spec((D, 8)), _full_spec((1, 8)), _full_spec((D, 8 * D)), _full_spec((8, D)),
        _full_spec((D, D)), _full_spec((1, D)), _full_spec((D, D)), _full_spec((1, D)),
        _full_spec((D, 2 * D)), _full_spec((1, 2 * D)), _full_spec((2 * D, D)), _full_spec((1, D)),
        _full_spec((D, 8)), _full_spec((1, 8)), _full_spec((D, 8 * D)), _full_spec((8, D)),
    ],
    out_specs=[
        pl.BlockSpec((BN, D), lambda i: (i, 0)),
        pl.BlockSpec((1, 1), lambda i: (0, 0)),
        pl.BlockSpec((1, 1), lambda i: (0, 0)),
    ],
    out_shape=[
        jax.ShapeDtypeStruct((N_NODES, D), jnp.float32),
        jax.ShapeDtypeStruct((1, 1), jnp.float32),
        jax.ShapeDtypeStruct((1, 1), jnp.float32),
    ],
)


_CB = 1280


def _add2_body(p_ref, o_ref):
    o_ref[...] = p_ref[0] + p_ref[1]


_add2_call = pl.pallas_call(
    _add2_body,
    grid=(NPAD // _CB,),
    in_specs=[pl.BlockSpec((2, _CB, D), lambda i: (0, i, 0))],
    out_specs=pl.BlockSpec((_CB, D), lambda i: (i, 0)),
    out_shape=jax.ShapeDtypeStruct((NPAD, D), jnp.float32),
)


def _finish_body(out1_ref, q_ref, tab_ref):
    s = out1_ref[...] + q_ref[0] + q_ref[1]
    tab_ref[...] = _expmap0(s, True)


_finish_call = pl.pallas_call(
    _finish_body,
    grid=(NPAD // _CB,),
    in_specs=[pl.BlockSpec((_CB, D), lambda i: (i, 0)),
              pl.BlockSpec((2, _CB, D), lambda i: (0, i, 0))],
    out_specs=pl.BlockSpec((_CB, D), lambda i: (i, 0)),
    out_shape=jax.ShapeDtypeStruct((NPAD, D), jnp.float32),
)


def _sc_mesh():
    return plsc.VectorSubcoreMesh(core_axis_name="c", subcore_axis_name="s",
                                  num_cores=NC, num_subcores=NS)


@functools.cache
def _spmm_call():
    return pl.kernel(
        _spmm_body,
        out_type=jax.ShapeDtypeStruct((NC, NPAD, D), jnp.float32),
        mesh=_sc_mesh(),
        compiler_params=pltpu.CompilerParams(needs_layout_passes=False),
        scratch_types=[
            pltpu.VMEM((2, CK), jnp.int32), pltpu.VMEM((2, CK), jnp.int32),
            pltpu.VMEM((2, CK), jnp.int32), pltpu.VMEM((2, CK), jnp.int32),
            pltpu.VMEM((CK,), jnp.float32), pltpu.VMEM((CK,), jnp.float32),
            pltpu.VMEM((CK,), jnp.float32), pltpu.VMEM((CK,), jnp.float32),
            pltpu.VMEM((CK, D), jnp.float32), pltpu.VMEM((CK, D), jnp.float32),
            pltpu.VMEM_SHARED((NPAD, D), jnp.float32),
            pltpu.SemaphoreType.DMA, pltpu.SemaphoreType.DMA,
            pltpu.SemaphoreType.DMA, pltpu.SemaphoreType.DMA,
            pltpu.SemaphoreType.DMA, pltpu.SemaphoreType.DMA,
            pltpu.SemaphoreType.DMA, pltpu.SemaphoreType.DMA,
        ],
    )


def _spmm_body(prev_hbm, cols_hbm, rows_hbm, w_hbm, out_hbm,
               e0, e1, e2, e3, w0, w1, w2, w3, gathA, gathB, acc,
               es0, es1, es2, es3, gs0, gs1, ss0, ss1):
    eb = (e0, e1, e2, e3)
    wb = (w0, w1, w2, w3)
    gth = (gathA, gathB)
    es = (es0, es1, es2, es3)
    gs = (gs0, gs1)
    ss = (ss0, ss1)
    cid = lax.axis_index("c")
    sid = lax.axis_index("s")
    wid = sid * NC + cid
    zero = jnp.zeros((16,), jnp.float32)

    def zrow(r, carry):
        for j in range(8):
            gathA[r, pl.ds(j * 16, 16)] = zero
        return carry

    lax.fori_loop(0, CK, zrow, 0)
    for k in range(RPT // CK):
        pltpu.sync_copy(gathA, acc.at[pl.ds(sid * RPT + k * CK, CK)])
    plsc.subcore_barrier()

    def fetch_edata(c, k, issue):
        pair = ((cols_hbm.at[wid, c], eb[k].at[0]),
                (rows_hbm.at[wid, c], eb[k].at[1]),
                (w_hbm.at[wid, c], wb[k]))
        for src, dst in pair:
            if issue:
                pltpu.async_copy(src, dst, es[k])
            else:
                pltpu.make_async_copy(src, dst, es[k]).wait()

    for e in range(3):
        fetch_edata(e, e, True)
    fetch_edata(0, 0, False)
    pltpu.async_copy(prev_hbm.at[eb[0].at[0]], gathA, gs[0])

    def outer(g, carry):
        for b in range(4):
            c = g * 4 + b
            gb = b % 2
            ogb = 1 - gb
            pltpu.make_async_copy(prev_hbm.at[eb[b].at[0]],
                                  gth[gb], gs[gb]).wait()

            def grp(g16, cc):
                wvec = wb[b][pl.ds(g16 * 16, 16)]
                for l in range(16):
                    e2_ = g16 * 16 + l
                    wgt = wvec[l]
                    for j in range(8):
                        sl = pl.ds(j * 16, 16)
                        gth[gb][e2_, sl] = gth[gb][e2_, sl] * wgt
                return cc

            lax.fori_loop(0, CK // 16, grp, 0)
            pltpu.async_copy(gth[gb], acc.at[eb[b].at[1]], ss[gb], add=True)

            @pl.when(c + 1 < NCH)
            def _():
                @pl.when(c >= 1)
                def _():
                    pltpu.make_async_copy(
                        gth[ogb], acc.at[eb[(b + 3) % 4].at[1]],
                        ss[ogb]).wait()
                fetch_edata(0, (b + 1) % 4, False)
                pltpu.async_copy(prev_hbm.at[eb[(b + 1) % 4].at[0]],
                                 gth[ogb], gs[ogb])

            @pl.when(c + 3 < NCH)
            def _():
                fetch_edata(c + 3, (b + 3) % 4, True)
        return carry

    lax.fori_loop(0, NCH // 4, outer, 0)
    pltpu.make_async_copy(gathB, acc.at[eb[3].at[1]], ss[1]).wait()
    plsc.subcore_barrier()

    pltpu.sync_copy(acc.at[pl.ds(sid * RPT, RPT)],
                    out_hbm.at[cid, pl.ds(sid * RPT, RPT)])


@functools.cache
def _warmup_call():
    return pl.kernel(
        _warmup_body,
        out_type=jax.ShapeDtypeStruct((NW, 16), jnp.float32),
        mesh=_sc_mesh(),
        compiler_params=pltpu.CompilerParams(needs_layout_passes=False),
        scratch_types=[pltpu.VMEM((16,), jnp.float32)],
    )


def _warmup_body(out_hbm, obuf):
    cid = lax.axis_index("c")
    sid = lax.axis_index("s")
    wid = sid * NC + cid
    obuf[...] = jnp.zeros((16,), jnp.float32)
    pltpu.sync_copy(obuf, out_hbm.at[wid])


@functools.cache
def _triple_call():
    return pl.kernel(
        _triple_body,
        out_type=jax.ShapeDtypeStruct((NW, 16), jnp.float32),
        mesh=_sc_mesh(),
        compiler_params=pltpu.CompilerParams(needs_layout_passes=False),
        scratch_types=[
            pltpu.VMEM((TPW,), jnp.int32),
            pltpu.VMEM((TPW,), jnp.int32),
            pltpu.VMEM((TPW,), jnp.int32),
            pltpu.VMEM((2, TCK, D), jnp.float32),
            pltpu.VMEM((2, TCK, D), jnp.float32),
            pltpu.VMEM((2, TCK, D), jnp.float32),
            pltpu.VMEM((16,), jnp.float32),
            pltpu.SemaphoreType.DMA, pltpu.SemaphoreType.DMA,
        ],
    )


def _triple_gathers(tab_hbm, av, pv, nv, ar, pr, nr, c, tb, sem, issue):
    copies = (
        (tab_hbm.at[av.at[pl.ds(c * TCK, TCK)]], ar.at[tb]),
        (tab_hbm.at[pv.at[pl.ds(c * TCK, TCK)]], pr.at[tb]),
        (tab_hbm.at[nv.at[pl.ds(c * TCK, TCK)]], nr.at[tb]),
    )
    for src, dst in copies:
        if issue:
            pltpu.async_copy(src, dst, sem)
        else:
            pltpu.make_async_copy(src, dst, sem).wait()


def _triple_body(tab_hbm, a_hbm, p_hbm, n_hbm, out_hbm,
                 av, pv, nv, ar, pr, nr, obuf, ts0, ts1):
    ts = (ts0, ts1)
    cid = lax.axis_index("c")
    sid = lax.axis_index("s")
    wid = sid * NC + cid
    pltpu.sync_copy(a_hbm.at[wid], av)
    pltpu.sync_copy(p_hbm.at[wid], pv)
    pltpu.sync_copy(n_hbm.at[wid], nv)

    _triple_gathers(tab_hbm, av, pv, nv, ar, pr, nr, 0, 0, ts[0], True)
    tot = jnp.zeros((16,), jnp.float32)
    for c in range(NTCH):
        tb = c % 2
        _triple_gathers(tab_hbm, av, pv, nv, ar, pr, nr, c, tb, ts[tb], False)
        if c + 1 < NTCH:
            _triple_gathers(tab_hbm, av, pv, nv, ar, pr, nr,
                            c + 1, 1 - tb, ts[1 - tb], True)

        def tri16(g, acc2):
            rows = lax.iota(jnp.int32, 16) + g * 16
            accp = jnp.zeros((16,), jnp.float32)
            accn = jnp.zeros((16,), jnp.float32)
            a0 = p0 = n0 = None
            for j in range(D):
                jv = jnp.full((16,), j, jnp.int32)
                a = plsc.load_gather(ar.at[tb], [rows, jv])
                pp = plsc.load_gather(pr.at[tb], [rows, jv])
                nn = plsc.load_gather(nr.at[tb], [rows, jv])
                if j == 0:
                    a0, p0, n0 = a, pp, nn
                accp = accp + a * pp
                accn = accn + a * nn
            ip = accp - 2.0 * a0 * p0
            inn = accn - 2.0 * a0 * n0
            dpos = jnp.maximum(-2.0 - 2.0 * ip, 0.0)
            dneg = jnp.maximum(-2.0 - 2.0 * inn, 0.0)
            return acc2 + jnp.maximum(dpos - dneg + MARGIN, 0.0)

        tot = lax.fori_loop(0, TCK // 16, tri16, tot)

    obuf[...] = tot
    pltpu.sync_copy(obuf, out_hbm.at[wid])


def kernel(edge_index, edge_weight, triples, eps, emb_user, emb_item,
           tr_gW, tr_gb, tr_eW, tr_eb, mu_W, mu_b, lv_W, lv_b,
           fc1_W, fc1_b, fc2_W, fc2_b, dec_gW, dec_gb, dec_eW, dec_eb):
    emb = jnp.concatenate([emb_user, emb_item], axis=0)
    tr_eWt = tr_eW.transpose(1, 0, 2).reshape(D, 8 * D)
    dec_eWt = dec_eW.transpose(1, 0, 2).reshape(D, 8 * D)
    m_ui, recon_sum, kl_sum = _dense_call(
        emb, eps,
        tr_gW, tr_gb.reshape(1, 8), tr_eWt, tr_eb,
        mu_W, mu_b.reshape(1, D), lv_W, lv_b.reshape(1, D),
        fc1_W, fc1_b.reshape(1, 2 * D), fc2_W, fc2_b.reshape(1, D),
        dec_gW, dec_gb.reshape(1, 8), dec_eWt, dec_eb,
    )
    recon_loss = recon_sum[0, 0] / N_NODES
    kl_loss = -0.5 * kl_sum[0, 0] / (N_NODES * D)

    pad = E_PAD - N_EDGES_IN
    row = jnp.pad(edge_index[0], (0, pad))
    col = jnp.pad(edge_index[1], (0, pad))
    warr = jnp.pad(edge_weight, (0, pad)).reshape(NW, NCH, CK)
    cols_w = col.reshape(NW, NCH, CK)
    rows_w = row.reshape(NW, NCH, CK)

    prev = jnp.pad(m_ui, ((0, NPAD - N_NODES), (0, 0)))
    spmm = _spmm_call()
    p = spmm(prev, cols_w, rows_w, warr)
    out1 = _add2_call(p)
    q = spmm(out1, cols_w, rows_w, warr)
    table = _finish_call(out1, q)

    a_w = triples[:, 0].reshape(NW, TPW)
    p_w = triples[:, 1].reshape(NW, TPW)
    n_w = triples[:, 2].reshape(NW, TPW)
    hinge = _triple_call()(table, a_w, p_w, n_w)
    margin_loss = jnp.sum(hinge)

    warm = _warmup_call()()
    return margin_loss + VAE_BETA * (recon_loss + kl_loss) + warm[0, 0]


# speedup vs baseline: 1.1825x; 1.1825x over previous
import functools

import jax
import jax.numpy as jnp
from jax import lax
from jax.experimental import pallas as pl
from jax.experimental.pallas import tpu as pltpu
from jax.experimental.pallas import tpu_sc as plsc

D = 128
N_NODES = 10000
NPAD = 10240
RPT = NPAD // 16
NC, NS = 2, 16
NW = NC * NS
N_EDGES_IN = 320000
E_PAD = 327680
EPW = E_PAD // NW
CK = 128
NCH = EPW // CK
NT = 16384
TPW = NT // NW
TCK = 128
NTCH = TPW // TCK
MARGIN = 0.1
VAE_BETA = 0.2
MAX_NORM = 1.5
BN = 1000
NBLK = N_NODES // BN


def _col0(x):
    return lax.broadcasted_iota(jnp.int32, x.shape, 1) == 0


def _logmap0(x):
    t = jnp.clip(x[:, :1], 1.0 + 1e-7, None)
    dist = jnp.log(t + jnp.sqrt(t * t - 1.0))
    n2 = jnp.maximum(jnp.sum(x * x, axis=-1, keepdims=True) - x[:, :1] * x[:, :1], 0.0)
    n = jnp.clip(jnp.sqrt(n2), 1e-7, None)
    return jnp.where(_col0(x), 0.0, x * (dist / n))


def _expmap0(u, project):
    n2 = jnp.maximum(jnp.sum(u * u, axis=-1, keepdims=True) - u[:, :1] * u[:, :1], 0.0)
    n = jnp.clip(jnp.sqrt(n2), 1e-7, None)
    nc = jnp.minimum(n, MAX_NORM) if project else n
    e = jnp.exp(nc)
    ei = 1.0 / e
    cosh = 0.5 * (e + ei)
    sinh = 0.5 * (e - ei)
    return jnp.where(_col0(u), cosh, u * (sinh / n))


def _sqdist(x, y):
    ip = jnp.sum(x * y, axis=-1) - 2.0 * x[:, 0] * y[:, 0]
    return jnp.maximum(-2.0 - 2.0 * ip, 0.0)


def _moe_block(x, gW, gb, eWt, eb):
    g = jnp.dot(x, gW, preferred_element_type=jnp.float32) + gb
    g = g - jnp.max(g, axis=-1, keepdims=True)
    w = jnp.exp(g)
    w = w / jnp.sum(w, axis=-1, keepdims=True)
    acc = jnp.zeros((x.shape[0], D), jnp.float32)
    for ex in range(8):
        ye = jnp.dot(x, eWt[:, ex * D:(ex + 1) * D], preferred_element_type=jnp.float32)
        acc = acc + w[:, ex:ex + 1] * (ye + eb[ex:ex + 1, :])
    return acc


def _encode_body(emb_ref, tr_gW, tr_gb, tr_eWt, tr_eb, mui_ref):
    x = emb_ref[...]
    t = _logmap0(x)
    mui_ref[...] = _moe_block(t, tr_gW[...], tr_gb[...], tr_eWt[...],
                              tr_eb[...])


def _vae_body(mui_ref, eps_ref, mu_W, mu_b, lv_W, lv_b,
              fc1_W, fc1_b, fc2_W, fc2_b, dec_gW, dec_gb, dec_eWt, dec_eb,
              recon_ref, kl_ref):
    m = mui_ref[...]
    mu = jnp.dot(m, mu_W[...], preferred_element_type=jnp.float32) + mu_b[...]
    lv = jnp.dot(m, lv_W[...], preferred_element_type=jnp.float32) + lv_b[...]
    std = jnp.exp(0.5 * lv)
    z = _expmap0(mu + eps_ref[...] * std, True)
    zt = _logmap0(z)
    h = jnp.maximum(jnp.dot(zt, fc1_W[...], preferred_element_type=jnp.float32) + fc1_b[...], 0.0)
    h = jnp.dot(h, fc2_W[...], preferred_element_type=jnp.float32) + fc2_b[...]
    h = _moe_block(h, dec_gW[...], dec_gb[...], dec_eWt[...], dec_eb[...])
    recon = _expmap0(h, True)
    target = _expmap0(m, False)
    rblk = jnp.sum(_sqdist(recon, target))
    kblk = jnp.sum(1.0 + lv - mu * mu - jnp.exp(lv))

    @pl.when(pl.program_id(0) == 0)
    def _init():
        recon_ref[...] = jnp.zeros_like(recon_ref)
        kl_ref[...] = jnp.zeros_like(kl_ref)

    recon_ref[...] = recon_ref[...] + rblk
    kl_ref[...] = kl_ref[...] + kblk


def _full_spec(shape):
    return pl.BlockSpec(shape, lambda i: tuple(0 for _ in shape))


_encode_call = pl.pallas_call(
    _encode_body,
    grid=(NBLK,),
    in_specs=[
        pl.BlockSpec((BN, D), lambda i: (i, 0)),
        _full_spec((D, 8)), _full_spec((1, 8)), _full_spec((D, 8 * D)), _full_spec((8, D)),
    ],
    out_specs=pl.BlockSpec((BN, D), lambda i: (i, 0)),
    out_shape=jax.ShapeDtypeStruct((N_NODES, D), jnp.float32),
)

_vae_call = pl.pallas_call(
    _vae_body,
    grid=(NBLK,),
    in_specs=[
        pl.BlockSpec((BN, D), lambda i: (i, 0)),
        pl.BlockSpec((BN, D), lambda i: (i, 0)),
        _full_spec((D, D)), _full_spec((1, D)), _full_spec((D, D)), _full_spec((1, D)),
        _full_spec((D, 2 * D)), _full_spec((1, 2 * D)), _full_spec((2 * D, D)), _full_spec((1, D)),
        _full_spec((D, 8)), _full_spec((1, 8)), _full_spec((D, 8 * D)), _full_spec((8, D)),
    ],
    out_specs=[
        pl.BlockSpec((1, 1), lambda i: (0, 0)),
        pl.BlockSpec((1, 1), lambda i: (0, 0)),
    ],
    out_shape=[
        jax.ShapeDtypeStruct((1, 1), jnp.float32),
        jax.ShapeDtypeStruct((1, 1), jnp.float32),
    ],
)


_CB = 1280


def _add2_body(p_ref, o_ref):
    o_ref[...] = p_ref[0] + p_ref[1]


_add2_call = pl.pallas_call(
    _add2_body,
    grid=(NPAD // _CB,),
    in_specs=[pl.BlockSpec((2, _CB, D), lambda i: (0, i, 0))],
    out_specs=pl.BlockSpec((_CB, D), lambda i: (i, 0)),
    out_shape=jax.ShapeDtypeStruct((NPAD, D), jnp.float32),
)


def _finish_body(out1_ref, q_ref, tab_ref):
    s = out1_ref[...] + q_ref[0] + q_ref[1]
    tab_ref[...] = _expmap0(s, True)


_finish_call = pl.pallas_call(
    _finish_body,
    grid=(NPAD // _CB,),
    in_specs=[pl.BlockSpec((_CB, D), lambda i: (i, 0)),
              pl.BlockSpec((2, _CB, D), lambda i: (0, i, 0))],
    out_specs=pl.BlockSpec((_CB, D), lambda i: (i, 0)),
    out_shape=jax.ShapeDtypeStruct((NPAD, D), jnp.float32),
)


def _sc_mesh():
    return plsc.VectorSubcoreMesh(core_axis_name="c", subcore_axis_name="s",
                                  num_cores=NC, num_subcores=NS)


@functools.cache
def _spmm_call():
    return pl.kernel(
        _spmm_body,
        out_type=jax.ShapeDtypeStruct((NC, NPAD, D), jnp.float32),
        mesh=_sc_mesh(),
        compiler_params=pltpu.CompilerParams(needs_layout_passes=False),
        scratch_types=[
            pltpu.VMEM((2, CK), jnp.int32), pltpu.VMEM((2, CK), jnp.int32),
            pltpu.VMEM((2, CK), jnp.int32), pltpu.VMEM((2, CK), jnp.int32),
            pltpu.VMEM((CK,), jnp.float32), pltpu.VMEM((CK,), jnp.float32),
            pltpu.VMEM((CK,), jnp.float32), pltpu.VMEM((CK,), jnp.float32),
            pltpu.VMEM((CK, D), jnp.float32), pltpu.VMEM((CK, D), jnp.float32),
            pltpu.VMEM_SHARED((NPAD, D), jnp.float32),
            pltpu.SemaphoreType.DMA, pltpu.SemaphoreType.DMA,
            pltpu.SemaphoreType.DMA, pltpu.SemaphoreType.DMA,
            pltpu.SemaphoreType.DMA, pltpu.SemaphoreType.DMA,
            pltpu.SemaphoreType.DMA, pltpu.SemaphoreType.DMA,
        ],
    )


def _spmm_body(prev_hbm, cols_hbm, rows_hbm, w_hbm, out_hbm,
               e0, e1, e2, e3, w0, w1, w2, w3, gathA, gathB, acc,
               es0, es1, es2, es3, gs0, gs1, ss0, ss1):
    eb = (e0, e1, e2, e3)
    wb = (w0, w1, w2, w3)
    gth = (gathA, gathB)
    es = (es0, es1, es2, es3)
    gs = (gs0, gs1)
    ss = (ss0, ss1)
    cid = lax.axis_index("c")
    sid = lax.axis_index("s")
    wid = sid * NC + cid
    zero = jnp.zeros((16,), jnp.float32)

    def zrow(r, carry):
        for j in range(8):
            gathA[r, pl.ds(j * 16, 16)] = zero
        return carry

    lax.fori_loop(0, CK, zrow, 0)
    for k in range(RPT // CK):
        pltpu.sync_copy(gathA, acc.at[pl.ds(sid * RPT + k * CK, CK)])
    plsc.subcore_barrier()

    def fetch_edata(c, k, issue):
        pair = ((cols_hbm.at[wid, c], eb[k].at[0]),
                (rows_hbm.at[wid, c], eb[k].at[1]),
                (w_hbm.at[wid, c], wb[k]))
        for src, dst in pair:
            if issue:
                pltpu.async_copy(src, dst, es[k])
            else:
                pltpu.make_async_copy(src, dst, es[k]).wait()

    for e in range(3):
        fetch_edata(e, e, True)
    fetch_edata(0, 0, False)
    pltpu.async_copy(prev_hbm.at[eb[0].at[0]], gathA, gs[0])

    def outer(g, carry):
        for b in range(4):
            c = g * 4 + b
            gb = b % 2
            ogb = 1 - gb
            pltpu.make_async_copy(prev_hbm.at[eb[b].at[0]],
                                  gth[gb], gs[gb]).wait()

            def grp(g16, cc):
                wvec = wb[b][pl.ds(g16 * 16, 16)]
                for l in range(16):
                    e2_ = g16 * 16 + l
                    wgt = wvec[l]
                    for j in range(8):
                        sl = pl.ds(j * 16, 16)
                        gth[gb][e2_, sl] = gth[gb][e2_, sl] * wgt
                return cc

            lax.fori_loop(0, CK // 16, grp, 0)
            pltpu.async_copy(gth[gb], acc.at[eb[b].at[1]], ss[gb], add=True)

            @pl.when(c + 1 < NCH)
            def _():
                @pl.when(c >= 1)
                def _():
                    pltpu.make_async_copy(
                        gth[ogb], acc.at[eb[(b + 3) % 4].at[1]],
                        ss[ogb]).wait()
                fetch_edata(0, (b + 1) % 4, False)
                pltpu.async_copy(prev_hbm.at[eb[(b + 1) % 4].at[0]],
                                 gth[ogb], gs[ogb])

            @pl.when(c + 3 < NCH)
            def _():
                fetch_edata(c + 3, (b + 3) % 4, True)
        return carry

    lax.fori_loop(0, NCH // 4, outer, 0)
    pltpu.make_async_copy(gathB, acc.at[eb[3].at[1]], ss[1]).wait()
    plsc.subcore_barrier()

    pltpu.sync_copy(acc.at[pl.ds(sid * RPT, RPT)],
                    out_hbm.at[cid, pl.ds(sid * RPT, RPT)])


@functools.cache
def _triple_call():
    return pl.kernel(
        _triple_body,
        out_type=jax.ShapeDtypeStruct((NW, 16), jnp.float32),
        mesh=_sc_mesh(),
        compiler_params=pltpu.CompilerParams(needs_layout_passes=False),
        scratch_types=[
            pltpu.VMEM((TPW,), jnp.int32),
            pltpu.VMEM((TPW,), jnp.int32),
            pltpu.VMEM((TPW,), jnp.int32),
            pltpu.VMEM((TCK, D), jnp.float32), pltpu.VMEM((TCK, D), jnp.float32),
            pltpu.VMEM((TCK, D), jnp.float32), pltpu.VMEM((TCK, D), jnp.float32),
            pltpu.VMEM((TCK, D), jnp.float32), pltpu.VMEM((TCK, D), jnp.float32),
            pltpu.VMEM((16,), jnp.float32),
            pltpu.SemaphoreType.DMA, pltpu.SemaphoreType.DMA,
        ],
    )


def _shuffle(v, idx):
    return lax.gather(
        v, idx[:, None],
        lax.GatherDimensionNumbers(offset_dims=(), collapsed_slice_dims=(0,),
                                   start_index_map=(0,)),
        (1,), mode=lax.GatherScatterMode.PROMISE_IN_BOUNDS)


def _triple_gathers(tab_hbm, idxs, bufs, c, sem, issue):
    for iv, dst in zip(idxs, bufs):
        src = tab_hbm.at[iv.at[pl.ds(c * TCK, TCK)]]
        if issue:
            pltpu.async_copy(src, dst, sem)
        else:
            pltpu.make_async_copy(src, dst, sem).wait()


def _triple_body(tab_hbm, a_hbm, p_hbm, n_hbm, out_hbm,
                 av, pv, nv, arA, arB, prA, prB, nrA, nrB, obuf, ts0, ts1):
    ts = (ts0, ts1)
    bufs = ((arA, prA, nrA), (arB, prB, nrB))
    idxs = (av, pv, nv)
    cid = lax.axis_index("c")
    sid = lax.axis_index("s")
    wid = sid * NC + cid
    pltpu.sync_copy(a_hbm.at[wid], av)
    pltpu.sync_copy(p_hbm.at[wid], pv)
    pltpu.sync_copy(n_hbm.at[wid], nv)

    _triple_gathers(tab_hbm, idxs, bufs[0], 0, ts[0], True)
    lanes = lax.iota(jnp.int32, 16)
    tot = jnp.float32(0.0)
    for c in range(NTCH):
        tb = c % 2
        A, P, N = bufs[tb]
        _triple_gathers(tab_hbm, idxs, bufs[tb], c, ts[tb], False)
        if c + 1 < NTCH:
            _triple_gathers(tab_hbm, idxs, bufs[1 - tb], c + 1,
                            ts[1 - tb], True)

        def tri(e, acc2):
            accp = jnp.zeros((16,), jnp.float32)
            accn = jnp.zeros((16,), jnp.float32)
            a0 = p0 = n0 = None
            for j in range(8):
                sl = pl.ds(j * 16, 16)
                a = A[e, sl]
                pp = P[e, sl]
                nn = N[e, sl]
                if j == 0:
                    a0, p0, n0 = a[0], pp[0], nn[0]
                accp = accp + a * pp
                accn = accn + a * nn
            for k in (1, 2, 4, 8):
                idx = lanes ^ k
                accp = accp + _shuffle(accp, idx)
                accn = accn + _shuffle(accn, idx)
            ip = accp[0] - 2.0 * a0 * p0
            inn = accn[0] - 2.0 * a0 * n0
            dpos = jnp.maximum(-2.0 - 2.0 * ip, 0.0)
            dneg = jnp.maximum(-2.0 - 2.0 * inn, 0.0)
            return acc2 + jnp.maximum(dpos - dneg + MARGIN, 0.0)

        tot = lax.fori_loop(0, TCK, tri, tot)

    obuf[...] = jnp.zeros((16,), jnp.float32) + tot
    pltpu.sync_copy(obuf, out_hbm.at[wid])


def kernel(edge_index, edge_weight, triples, eps, emb_user, emb_item,
           tr_gW, tr_gb, tr_eW, tr_eb, mu_W, mu_b, lv_W, lv_b,
           fc1_W, fc1_b, fc2_W, fc2_b, dec_gW, dec_gb, dec_eW, dec_eb):
    emb = jnp.concatenate([emb_user, emb_item], axis=0)
    tr_eWt = tr_eW.transpose(1, 0, 2).reshape(D, 8 * D)
    dec_eWt = dec_eW.transpose(1, 0, 2).reshape(D, 8 * D)
    m_ui = _encode_call(emb, tr_gW, tr_gb.reshape(1, 8), tr_eWt, tr_eb)
    recon_sum, kl_sum = _vae_call(
        m_ui, eps,
        mu_W, mu_b.reshape(1, D), lv_W, lv_b.reshape(1, D),
        fc1_W, fc1_b.reshape(1, 2 * D), fc2_W, fc2_b.reshape(1, D),
        dec_gW, dec_gb.reshape(1, 8), dec_eWt, dec_eb,
    )
    recon_loss = recon_sum[0, 0] / N_NODES
    kl_loss = -0.5 * kl_sum[0, 0] / (N_NODES * D)

    pad = E_PAD - N_EDGES_IN
    row = jnp.pad(edge_index[0], (0, pad))
    col = jnp.pad(edge_index[1], (0, pad))
    warr = jnp.pad(edge_weight, (0, pad)).reshape(NW, NCH, CK)
    cols_w = col.reshape(NW, NCH, CK)
    rows_w = row.reshape(NW, NCH, CK)

    prev = jnp.pad(m_ui, ((0, NPAD - N_NODES), (0, 0)))
    spmm = _spmm_call()
    p = spmm(prev, cols_w, rows_w, warr)
    out1 = _add2_call(p)
    q = spmm(out1, cols_w, rows_w, warr)
    table = _finish_call(out1, q)

    a_w = triples[:, 0].reshape(NW, TPW)
    p_w = triples[:, 1].reshape(NW, TPW)
    n_w = triples[:, 2].reshape(NW, TPW)
    hinge = _triple_call()(table, a_w, p_w, n_w)
    margin_loss = jnp.sum(hinge)

    return margin_loss + VAE_BETA * (recon_loss + kl_loss)


# speedup vs baseline: 1.1826x; 1.0001x over previous
import functools

import jax
import jax.numpy as jnp
from jax import lax
from jax.experimental import pallas as pl
from jax.experimental.pallas import tpu as pltpu
from jax.experimental.pallas import tpu_sc as plsc

D = 128
N_NODES = 10000
NPAD = 10240
RPT = NPAD // 16
NC, NS = 2, 16
NW = NC * NS
N_EDGES_IN = 320000
E_PAD = 327680
EPW = E_PAD // NW
CK = 128
NCH = EPW // CK
NT = 16384
TPW = NT // NW
TCK = 128
NTCH = TPW // TCK
MARGIN = 0.1
VAE_BETA = 0.2
MAX_NORM = 1.5
BN = 1000
NBLK = N_NODES // BN


def _col0(x):
    return lax.broadcasted_iota(jnp.int32, x.shape, 1) == 0


def _logmap0(x):
    t = jnp.clip(x[:, :1], 1.0 + 1e-7, None)
    dist = jnp.log(t + jnp.sqrt(t * t - 1.0))
    n2 = jnp.maximum(jnp.sum(x * x, axis=-1, keepdims=True) - x[:, :1] * x[:, :1], 0.0)
    n = jnp.clip(jnp.sqrt(n2), 1e-7, None)
    return jnp.where(_col0(x), 0.0, x * (dist / n))


def _expmap0(u, project):
    n2 = jnp.maximum(jnp.sum(u * u, axis=-1, keepdims=True) - u[:, :1] * u[:, :1], 0.0)
    n = jnp.clip(jnp.sqrt(n2), 1e-7, None)
    nc = jnp.minimum(n, MAX_NORM) if project else n
    e = jnp.exp(nc)
    ei = 1.0 / e
    cosh = 0.5 * (e + ei)
    sinh = 0.5 * (e - ei)
    return jnp.where(_col0(u), cosh, u * (sinh / n))


def _sqdist(x, y):
    ip = jnp.sum(x * y, axis=-1) - 2.0 * x[:, 0] * y[:, 0]
    return jnp.maximum(-2.0 - 2.0 * ip, 0.0)


def _moe_block(x, gW, gb, eWt, eb):
    g = jnp.dot(x, gW, preferred_element_type=jnp.float32) + gb
    g = g - jnp.max(g, axis=-1, keepdims=True)
    w = jnp.exp(g)
    w = w / jnp.sum(w, axis=-1, keepdims=True)
    acc = jnp.zeros((x.shape[0], D), jnp.float32)
    for ex in range(8):
        ye = jnp.dot(x, eWt[:, ex * D:(ex + 1) * D], preferred_element_type=jnp.float32)
        acc = acc + w[:, ex:ex + 1] * (ye + eb[ex:ex + 1, :])
    return acc


def _encode_body(emb_ref, tr_gW, tr_gb, tr_eWt, tr_eb, mui_ref):
    x = emb_ref[...]
    t = _logmap0(x)
    mui_ref[...] = _moe_block(t, tr_gW[...], tr_gb[...], tr_eWt[...],
                              tr_eb[...])


def _vae_body(mui_ref, eps_ref, mu_W, mu_b, lv_W, lv_b,
              fc1_W, fc1_b, fc2_W, fc2_b, dec_gW, dec_gb, dec_eWt, dec_eb,
              recon_ref, kl_ref):
    m = mui_ref[...]
    mu = jnp.dot(m, mu_W[...], preferred_element_type=jnp.float32) + mu_b[...]
    lv = jnp.dot(m, lv_W[...], preferred_element_type=jnp.float32) + lv_b[...]
    std = jnp.exp(0.5 * lv)
    z = _expmap0(mu + eps_ref[...] * std, True)
    zt = _logmap0(z)
    h = jnp.maximum(jnp.dot(zt, fc1_W[...], preferred_element_type=jnp.float32) + fc1_b[...], 0.0)
    h = jnp.dot(h, fc2_W[...], preferred_element_type=jnp.float32) + fc2_b[...]
    h = _moe_block(h, dec_gW[...], dec_gb[...], dec_eWt[...], dec_eb[...])
    recon = _expmap0(h, True)
    target = _expmap0(m, False)
    rblk = jnp.sum(_sqdist(recon, target))
    kblk = jnp.sum(1.0 + lv - mu * mu - jnp.exp(lv))

    @pl.when(pl.program_id(0) == 0)
    def _init():
        recon_ref[...] = jnp.zeros_like(recon_ref)
        kl_ref[...] = jnp.zeros_like(kl_ref)

    recon_ref[...] = recon_ref[...] + rblk
    kl_ref[...] = kl_ref[...] + kblk


def _full_spec(shape):
    return pl.BlockSpec(shape, lambda i: tuple(0 for _ in shape))


_encode_call = pl.pallas_call(
    _encode_body,
    grid=(NBLK,),
    in_specs=[
        pl.BlockSpec((BN, D), lambda i: (i, 0)),
        _full_spec((D, 8)), _full_spec((1, 8)), _full_spec((D, 8 * D)), _full_spec((8, D)),
    ],
    out_specs=pl.BlockSpec((BN, D), lambda i: (i, 0)),
    out_shape=jax.ShapeDtypeStruct((N_NODES, D), jnp.float32),
)

_vae_call = pl.pallas_call(
    _vae_body,
    grid=(NBLK,),
    in_specs=[
        pl.BlockSpec((BN, D), lambda i: (i, 0)),
        pl.BlockSpec((BN, D), lambda i: (i, 0)),
        _full_spec((D, D)), _full_spec((1, D)), _full_spec((D, D)), _full_spec((1, D)),
        _full_spec((D, 2 * D)), _full_spec((1, 2 * D)), _full_spec((2 * D, D)), _full_spec((1, D)),
        _full_spec((D, 8)), _full_spec((1, 8)), _full_spec((D, 8 * D)), _full_spec((8, D)),
    ],
    out_specs=[
        pl.BlockSpec((1, 1), lambda i: (0, 0)),
        pl.BlockSpec((1, 1), lambda i: (0, 0)),
    ],
    out_shape=[
        jax.ShapeDtypeStruct((1, 1), jnp.float32),
        jax.ShapeDtypeStruct((1, 1), jnp.float32),
    ],
)


_CB = 1280


def _add2_body(p_ref, o_ref):
    o_ref[...] = p_ref[0] + p_ref[1]


_add2_call = pl.pallas_call(
    _add2_body,
    grid=(NPAD // _CB,),
    in_specs=[pl.BlockSpec((2, _CB, D), lambda i: (0, i, 0))],
    out_specs=pl.BlockSpec((_CB, D), lambda i: (i, 0)),
    out_shape=jax.ShapeDtypeStruct((NPAD, D), jnp.float32),
)


def _finish_body(out1_ref, q_ref, tab_ref):
    s = out1_ref[...] + q_ref[0] + q_ref[1]
    tab_ref[...] = _expmap0(s, True)


_finish_call = pl.pallas_call(
    _finish_body,
    grid=(NPAD // _CB,),
    in_specs=[pl.BlockSpec((_CB, D), lambda i: (i, 0)),
              pl.BlockSpec((2, _CB, D), lambda i: (0, i, 0))],
    out_specs=pl.BlockSpec((_CB, D), lambda i: (i, 0)),
    out_shape=jax.ShapeDtypeStruct((NPAD, D), jnp.float32),
)


def _sc_mesh():
    return plsc.VectorSubcoreMesh(core_axis_name="c", subcore_axis_name="s",
                                  num_cores=NC, num_subcores=NS)


@functools.cache
def _spmm_call():
    return pl.kernel(
        _spmm_body,
        out_type=jax.ShapeDtypeStruct((NC, NPAD, D), jnp.float32),
        mesh=_sc_mesh(),
        compiler_params=pltpu.CompilerParams(needs_layout_passes=False),
        scratch_types=[
            pltpu.VMEM((2, CK), jnp.int32), pltpu.VMEM((2, CK), jnp.int32),
            pltpu.VMEM((2, CK), jnp.int32), pltpu.VMEM((2, CK), jnp.int32),
            pltpu.VMEM((CK,), jnp.float32), pltpu.VMEM((CK,), jnp.float32),
            pltpu.VMEM((CK,), jnp.float32), pltpu.VMEM((CK,), jnp.float32),
            pltpu.VMEM((CK, D), jnp.float32), pltpu.VMEM((CK, D), jnp.float32),
            pltpu.VMEM_SHARED((NPAD, D), jnp.float32),
            pltpu.SemaphoreType.DMA, pltpu.SemaphoreType.DMA,
            pltpu.SemaphoreType.DMA, pltpu.SemaphoreType.DMA,
            pltpu.SemaphoreType.DMA, pltpu.SemaphoreType.DMA,
            pltpu.SemaphoreType.DMA, pltpu.SemaphoreType.DMA,
        ],
    )


def _spmm_body(prev_hbm, cols_hbm, rows_hbm, w_hbm, out_hbm,
               e0, e1, e2, e3, w0, w1, w2, w3, gathA, gathB, acc,
               es0, es1, es2, es3, gs0, gs1, ss0, ss1):
    eb = (e0, e1, e2, e3)
    wb = (w0, w1, w2, w3)
    gth = (gathA, gathB)
    es = (es0, es1, es2, es3)
    gs = (gs0, gs1)
    ss = (ss0, ss1)
    cid = lax.axis_index("c")
    sid = lax.axis_index("s")
    wid = sid * NC + cid
    zero = jnp.zeros((16,), jnp.float32)

    def zrow(r, carry):
        for j in range(8):
            gathA[r, pl.ds(j * 16, 16)] = zero
        return carry

    lax.fori_loop(0, CK, zrow, 0)
    for k in range(RPT // CK):
        pltpu.sync_copy(gathA, acc.at[pl.ds(sid * RPT + k * CK, CK)])
    plsc.subcore_barrier()

    def fetch_edata(c, k, issue):
        pair = ((cols_hbm.at[wid, c], eb[k].at[0]),
                (rows_hbm.at[wid, c], eb[k].at[1]),
                (w_hbm.at[wid, c], wb[k]))
        for src, dst in pair:
            if issue:
                pltpu.async_copy(src, dst, es[k])
            else:
                pltpu.make_async_copy(src, dst, es[k]).wait()

    for e in range(3):
        fetch_edata(e, e, True)
    fetch_edata(0, 0, False)
    pltpu.async_copy(prev_hbm.at[eb[0].at[0]], gathA, gs[0])

    def outer(g, carry):
        for b in range(4):
            c = g * 4 + b
            gb = b % 2
            ogb = 1 - gb
            pltpu.make_async_copy(prev_hbm.at[eb[b].at[0]],
                                  gth[gb], gs[gb]).wait()

            def grp(g16, cc):
                wvec = wb[b][pl.ds(g16 * 16, 16)]
                for l in range(16):
                    e2_ = g16 * 16 + l
                    wgt = wvec[l]
                    for j in range(8):
                        sl = pl.ds(j * 16, 16)
                        gth[gb][e2_, sl] = gth[gb][e2_, sl] * wgt
                return cc

            lax.fori_loop(0, CK // 16, grp, 0)
            pltpu.async_copy(gth[gb], acc.at[eb[b].at[1]], ss[gb], add=True)

            @pl.when(c + 1 < NCH)
            def _():
                @pl.when(c >= 1)
                def _():
                    pltpu.make_async_copy(
                        gth[ogb], acc.at[eb[(b + 3) % 4].at[1]],
                        ss[ogb]).wait()
                fetch_edata(0, (b + 1) % 4, False)
                pltpu.async_copy(prev_hbm.at[eb[(b + 1) % 4].at[0]],
                                 gth[ogb], gs[ogb])

            @pl.when(c + 3 < NCH)
            def _():
                fetch_edata(c + 3, (b + 3) % 4, True)
        return carry

    lax.fori_loop(0, NCH // 4, outer, 0)
    pltpu.make_async_copy(gathB, acc.at[eb[3].at[1]], ss[1]).wait()
    plsc.subcore_barrier()

    pltpu.sync_copy(acc.at[pl.ds(sid * RPT, RPT)],
                    out_hbm.at[cid, pl.ds(sid * RPT, RPT)])


@functools.cache
def _triple_call():
    return pl.kernel(
        _triple_body,
        out_type=jax.ShapeDtypeStruct((NW, 16), jnp.float32),
        mesh=_sc_mesh(),
        compiler_params=pltpu.CompilerParams(needs_layout_passes=False),
        scratch_types=[
            pltpu.VMEM((TPW,), jnp.int32),
            pltpu.VMEM((TPW,), jnp.int32),
            pltpu.VMEM((TPW,), jnp.int32),
            pltpu.VMEM((TCK, D), jnp.float32), pltpu.VMEM((TCK, D), jnp.float32),
            pltpu.VMEM((TCK, D), jnp.float32), pltpu.VMEM((TCK, D), jnp.float32),
            pltpu.VMEM((TCK, D), jnp.float32), pltpu.VMEM((TCK, D), jnp.float32),
            pltpu.VMEM((16,), jnp.float32),
            pltpu.SemaphoreType.DMA, pltpu.SemaphoreType.DMA,
        ],
    )


def _shuffle(v, idx):
    return lax.gather(
        v, idx[:, None],
        lax.GatherDimensionNumbers(offset_dims=(), collapsed_slice_dims=(0,),
                                   start_index_map=(0,)),
        (1,), mode=lax.GatherScatterMode.PROMISE_IN_BOUNDS)


def _triple_gathers(tab_hbm, idxs, bufs, c, sem, issue):
    for iv, dst in zip(idxs, bufs):
        src = tab_hbm.at[iv.at[pl.ds(c * TCK, TCK)]]
        if issue:
            pltpu.async_copy(src, dst, sem)
        else:
            pltpu.make_async_copy(src, dst, sem).wait()


def _triple_body(tab_hbm, a_hbm, p_hbm, n_hbm, out_hbm,
                 av, pv, nv, arA, arB, prA, prB, nrA, nrB, obuf, ts0, ts1):
    ts = (ts0, ts1)
    bufs = ((arA, prA, nrA), (arB, prB, nrB))
    idxs = (av, pv, nv)
    cid = lax.axis_index("c")
    sid = lax.axis_index("s")
    wid = sid * NC + cid
    pltpu.sync_copy(a_hbm.at[wid], av)
    pltpu.sync_copy(p_hbm.at[wid], pv)
    pltpu.sync_copy(n_hbm.at[wid], nv)

    _triple_gathers(tab_hbm, idxs, bufs[0], 0, ts[0], True)
    lanes = lax.iota(jnp.int32, 16)
    tot = jnp.float32(0.0)
    for c in range(NTCH):
        tb = c % 2
        A, P, N = bufs[tb]
        _triple_gathers(tab_hbm, idxs, bufs[tb], c, ts[tb], False)
        if c + 1 < NTCH:
            _triple_gathers(tab_hbm, idxs, bufs[1 - tb], c + 1,
                            ts[1 - tb], True)

        def tri(e, acc2):
            accp = jnp.zeros((16,), jnp.float32)
            accn = jnp.zeros((16,), jnp.float32)
            a0 = p0 = n0 = None
            for j in range(8):
                sl = pl.ds(j * 16, 16)
                a = A[e, sl]
                pp = P[e, sl]
                nn = N[e, sl]
                if j == 0:
                    a0, p0, n0 = a[0], pp[0], nn[0]
                accp = accp + a * pp
                accn = accn + a * nn
            for k in (1, 2, 4, 8):
                idx = lanes ^ k
                accp = accp + _shuffle(accp, idx)
                accn = accn + _shuffle(accn, idx)
            ip = accp[0] - 2.0 * a0 * p0
            inn = accn[0] - 2.0 * a0 * n0
            dpos = jnp.maximum(-2.0 - 2.0 * ip, 0.0)
            dneg = jnp.maximum(-2.0 - 2.0 * inn, 0.0)
            return acc2 + jnp.maximum(dpos - dneg + MARGIN, 0.0)

        tot = lax.fori_loop(0, TCK, tri, tot)

    obuf[...] = jnp.zeros((16,), jnp.float32) + tot
    pltpu.sync_copy(obuf, out_hbm.at[wid])


def kernel(edge_index, edge_weight, triples, eps, emb_user, emb_item,
           tr_gW, tr_gb, tr_eW, tr_eb, mu_W, mu_b, lv_W, lv_b,
           fc1_W, fc1_b, fc2_W, fc2_b, dec_gW, dec_gb, dec_eW, dec_eb):
    emb = jnp.concatenate([emb_user, emb_item], axis=0)
    tr_eWt = tr_eW.transpose(1, 0, 2).reshape(D, 8 * D)
    dec_eWt = dec_eW.transpose(1, 0, 2).reshape(D, 8 * D)
    m_ui = _encode_call(emb, tr_gW, tr_gb.reshape(1, 8), tr_eWt, tr_eb)
    recon_sum, kl_sum = _vae_call(
        m_ui, eps,
        mu_W, mu_b.reshape(1, D), lv_W, lv_b.reshape(1, D),
        fc1_W, fc1_b.reshape(1, 2 * D), fc2_W, fc2_b.reshape(1, D),
        dec_gW, dec_gb.reshape(1, 8), dec_eWt, dec_eb,
    )
    recon_loss = recon_sum[0, 0] / N_NODES
    kl_loss = -0.5 * kl_sum[0, 0] / (N_NODES * D)

    pad = E_PAD - N_EDGES_IN
    row = jnp.pad(edge_index[0], (0, pad))
    col = jnp.pad(edge_index[1], (0, pad))
    warr = jnp.pad(edge_weight, (0, pad)).reshape(NW, NCH, CK)
    cols_w = col.reshape(NW, NCH, CK)
    rows_w = row.reshape(NW, NCH, CK)

    prev = jnp.pad(m_ui, ((0, NPAD - N_NODES), (0, 0)))
    spmm = _spmm_call()
    p = spmm(prev, cols_w, rows_w, warr)
    out1 = _add2_call(p)
    q = spmm(out1, cols_w, rows_w, warr)
    table = _finish_call(out1, q)

    a_w = triples[:, 0].reshape(NW, TPW)
    p_w = triples[:, 1].reshape(NW, TPW)
    n_w = triples[:, 2].reshape(NW, TPW)
    hinge = _triple_call()(table, a_w, p_w, n_w)
    margin_loss = jnp.sum(hinge[:, 0])

    return margin_loss + VAE_BETA * (recon_loss + kl_loss)


# speedup vs baseline: 1.1827x; 1.0001x over previous
import functools

import jax
import jax.numpy as jnp
from jax import lax
from jax.experimental import pallas as pl
from jax.experimental.pallas import tpu as pltpu
from jax.experimental.pallas import tpu_sc as plsc

D = 128
N_NODES = 10000
NPAD = 10240
RPT = NPAD // 16
NC, NS = 2, 16
NW = NC * NS
N_EDGES_IN = 320000
E_PAD = 327680
EPW = E_PAD // NW
CK = 128
NCH = EPW // CK
NT = 16384
TPW = NT // NW
TCK = 128
NTCH = TPW // TCK
MARGIN = 0.1
VAE_BETA = 0.2
MAX_NORM = 1.5
BN = 1000
NBLK = N_NODES // BN


def _col0(x):
    return lax.broadcasted_iota(jnp.int32, x.shape, 1) == 0


def _logmap0(x):
    t = jnp.clip(x[:, :1], 1.0 + 1e-7, None)
    dist = jnp.log(t + jnp.sqrt(t * t - 1.0))
    n2 = jnp.maximum(jnp.sum(x * x, axis=-1, keepdims=True) - x[:, :1] * x[:, :1], 0.0)
    n = jnp.clip(jnp.sqrt(n2), 1e-7, None)
    return jnp.where(_col0(x), 0.0, x * (dist / n))


def _expmap0(u, project):
    n2 = jnp.maximum(jnp.sum(u * u, axis=-1, keepdims=True) - u[:, :1] * u[:, :1], 0.0)
    n = jnp.clip(jnp.sqrt(n2), 1e-7, None)
    nc = jnp.minimum(n, MAX_NORM) if project else n
    e = jnp.exp(nc)
    ei = 1.0 / e
    cosh = 0.5 * (e + ei)
    sinh = 0.5 * (e - ei)
    return jnp.where(_col0(u), cosh, u * (sinh / n))


def _sqdist(x, y):
    ip = jnp.sum(x * y, axis=-1) - 2.0 * x[:, 0] * y[:, 0]
    return jnp.maximum(-2.0 - 2.0 * ip, 0.0)


def _moe_block(x, gW, gb, eWt, eb):
    g = jnp.dot(x, gW, preferred_element_type=jnp.float32) + gb
    g = g - jnp.max(g, axis=-1, keepdims=True)
    w = jnp.exp(g)
    w = w / jnp.sum(w, axis=-1, keepdims=True)
    acc = jnp.zeros((x.shape[0], D), jnp.float32)
    for ex in range(8):
        ye = jnp.dot(x, eWt[:, ex * D:(ex + 1) * D], preferred_element_type=jnp.float32)
        acc = acc + w[:, ex:ex + 1] * (ye + eb[ex:ex + 1, :])
    return acc


def _encode_body(emb_ref, tr_gW, tr_gb, tr_eWt, tr_eb, mui_ref):
    x = emb_ref[...]
    t = _logmap0(x)
    mui_ref[...] = _moe_block(t, tr_gW[...], tr_gb[...], tr_eWt[...],
                              tr_eb[...])


def _vae_body(mui_ref, eps_ref, mu_W, mu_b, lv_W, lv_b,
              fc1_W, fc1_b, fc2_W, fc2_b, dec_gW, dec_gb, dec_eWt, dec_eb,
              recon_ref, kl_ref):
    m = mui_ref[...]
    mu = jnp.dot(m, mu_W[...], preferred_element_type=jnp.float32) + mu_b[...]
    lv = jnp.dot(m, lv_W[...], preferred_element_type=jnp.float32) + lv_b[...]
    std = jnp.exp(0.5 * lv)
    z = _expmap0(mu + eps_ref[...] * std, True)
    zt = _logmap0(z)
    h = jnp.maximum(jnp.dot(zt, fc1_W[...], preferred_element_type=jnp.float32) + fc1_b[...], 0.0)
    h = jnp.dot(h, fc2_W[...], preferred_element_type=jnp.float32) + fc2_b[...]
    h = _moe_block(h, dec_gW[...], dec_gb[...], dec_eWt[...], dec_eb[...])
    recon = _expmap0(h, True)
    target = _expmap0(m, False)
    rblk = jnp.sum(_sqdist(recon, target))
    kblk = jnp.sum(1.0 + lv - mu * mu - jnp.exp(lv))

    @pl.when(pl.program_id(0) == 0)
    def _init():
        recon_ref[...] = jnp.zeros_like(recon_ref)
        kl_ref[...] = jnp.zeros_like(kl_ref)

    recon_ref[...] = recon_ref[...] + rblk
    kl_ref[...] = kl_ref[...] + kblk


def _full_spec(shape):
    return pl.BlockSpec(shape, lambda i: tuple(0 for _ in shape))


_encode_call = pl.pallas_call(
    _encode_body,
    grid=(NBLK,),
    in_specs=[
        pl.BlockSpec((BN, D), lambda i: (i, 0)),
        _full_spec((D, 8)), _full_spec((1, 8)), _full_spec((D, 8 * D)), _full_spec((8, D)),
    ],
    out_specs=pl.BlockSpec((BN, D), lambda i: (i, 0)),
    out_shape=jax.ShapeDtypeStruct((N_NODES, D), jnp.float32),
)

_vae_call = pl.pallas_call(
    _vae_body,
    grid=(NBLK,),
    in_specs=[
        pl.BlockSpec((BN, D), lambda i: (i, 0)),
        pl.BlockSpec((BN, D), lambda i: (i, 0)),
        _full_spec((D, D)), _full_spec((1, D)), _full_spec((D, D)), _full_spec((1, D)),
        _full_spec((D, 2 * D)), _full_spec((1, 2 * D)), _full_spec((2 * D, D)), _full_spec((1, D)),
        _full_spec((D, 8)), _full_spec((1, 8)), _full_spec((D, 8 * D)), _full_spec((8, D)),
    ],
    out_specs=[
        pl.BlockSpec((1, 1), lambda i: (0, 0)),
        pl.BlockSpec((1, 1), lambda i: (0, 0)),
    ],
    out_shape=[
        jax.ShapeDtypeStruct((1, 1), jnp.float32),
        jax.ShapeDtypeStruct((1, 1), jnp.float32),
    ],
)


_CB = 1280


def _add2_body(p_ref, o_ref):
    o_ref[...] = p_ref[0] + p_ref[1]


_add2_call = pl.pallas_call(
    _add2_body,
    grid=(NPAD // _CB,),
    in_specs=[pl.BlockSpec((2, _CB, D), lambda i: (0, i, 0))],
    out_specs=pl.BlockSpec((_CB, D), lambda i: (i, 0)),
    out_shape=jax.ShapeDtypeStruct((NPAD, D), jnp.float32),
)


def _finish_body(out1_ref, q_ref, tab_ref):
    s = out1_ref[...] + q_ref[0] + q_ref[1]
    tab_ref[...] = _expmap0(s, True)


_finish_call = pl.pallas_call(
    _finish_body,
    grid=(NPAD // _CB,),
    in_specs=[pl.BlockSpec((_CB, D), lambda i: (i, 0)),
              pl.BlockSpec((2, _CB, D), lambda i: (0, i, 0))],
    out_specs=pl.BlockSpec((_CB, D), lambda i: (i, 0)),
    out_shape=jax.ShapeDtypeStruct((NPAD, D), jnp.float32),
)


def _sc_mesh():
    return plsc.VectorSubcoreMesh(core_axis_name="c", subcore_axis_name="s",
                                  num_cores=NC, num_subcores=NS)


@functools.cache
def _spmm_call():
    return pl.kernel(
        _spmm_body,
        out_type=jax.ShapeDtypeStruct((NC, NPAD, D), jnp.float32),
        mesh=_sc_mesh(),
        compiler_params=pltpu.CompilerParams(needs_layout_passes=False),
        scratch_types=[
            pltpu.VMEM((2, CK), jnp.int32), pltpu.VMEM((2, CK), jnp.int32),
            pltpu.VMEM((2, CK), jnp.int32), pltpu.VMEM((2, CK), jnp.int32),
            pltpu.VMEM((CK,), jnp.float32), pltpu.VMEM((CK,), jnp.float32),
            pltpu.VMEM((CK,), jnp.float32), pltpu.VMEM((CK,), jnp.float32),
            pltpu.VMEM((CK, D), jnp.float32), pltpu.VMEM((CK, D), jnp.float32),
            pltpu.VMEM_SHARED((NPAD, D), jnp.float32),
            pltpu.SemaphoreType.DMA, pltpu.SemaphoreType.DMA,
            pltpu.SemaphoreType.DMA, pltpu.SemaphoreType.DMA,
            pltpu.SemaphoreType.DMA, pltpu.SemaphoreType.DMA,
            pltpu.SemaphoreType.DMA, pltpu.SemaphoreType.DMA,
        ],
    )


def _spmm_body(prev_hbm, cols_hbm, rows_hbm, w_hbm, out_hbm,
               e0, e1, e2, e3, w0, w1, w2, w3, gathA, gathB, acc,
               es0, es1, es2, es3, gs0, gs1, ss0, ss1):
    eb = (e0, e1, e2, e3)
    wb = (w0, w1, w2, w3)
    gth = (gathA, gathB)
    es = (es0, es1, es2, es3)
    gs = (gs0, gs1)
    ss = (ss0, ss1)
    cid = lax.axis_index("c")
    sid = lax.axis_index("s")
    wid = sid * NC + cid
    zero = jnp.zeros((16,), jnp.float32)

    def zrow(r, carry):
        for j in range(8):
            gathA[r, pl.ds(j * 16, 16)] = zero
        return carry

    lax.fori_loop(0, CK, zrow, 0)
    for k in range(RPT // CK):
        pltpu.sync_copy(gathA, acc.at[pl.ds(sid * RPT + k * CK, CK)])
    plsc.subcore_barrier()

    def fetch_edata(c, k, issue):
        pair = ((cols_hbm.at[wid, c], eb[k].at[0]),
                (rows_hbm.at[wid, c], eb[k].at[1]),
                (w_hbm.at[wid, c], wb[k]))
        for src, dst in pair:
            if issue:
                pltpu.async_copy(src, dst, es[k])
            else:
                pltpu.make_async_copy(src, dst, es[k]).wait()

    for e in range(3):
        fetch_edata(e, e, True)
    fetch_edata(0, 0, False)
    pltpu.async_copy(prev_hbm.at[eb[0].at[0]], gathA, gs[0])

    def outer(g, carry):
        for b in range(4):
            c = g * 4 + b
            gb = b % 2
            ogb = 1 - gb
            pltpu.make_async_copy(prev_hbm.at[eb[b].at[0]],
                                  gth[gb], gs[gb]).wait()

            def grp(g16, cc):
                wvec = wb[b][pl.ds(g16 * 16, 16)]
                for l in range(16):
                    e2_ = g16 * 16 + l
                    wgt = wvec[l]
                    for j in range(8):
                        sl = pl.ds(j * 16, 16)
                        gth[gb][e2_, sl] = gth[gb][e2_, sl] * wgt
                return cc

            lax.fori_loop(0, CK // 16, grp, 0)
            pltpu.async_copy(gth[gb], acc.at[eb[b].at[1]], ss[gb], add=True)

            @pl.when(c + 1 < NCH)
            def _():
                @pl.when(c >= 1)
                def _():
                    pltpu.make_async_copy(
                        gth[ogb], acc.at[eb[(b + 3) % 4].at[1]],
                        ss[ogb]).wait()
                fetch_edata(0, (b + 1) % 4, False)
                pltpu.async_copy(prev_hbm.at[eb[(b + 1) % 4].at[0]],
                                 gth[ogb], gs[ogb])

            @pl.when(c + 3 < NCH)
            def _():
                fetch_edata(c + 3, (b + 3) % 4, True)
        return carry

    lax.fori_loop(0, NCH // 4, outer, 0)
    pltpu.make_async_copy(gathB, acc.at[eb[3].at[1]], ss[1]).wait()
    plsc.subcore_barrier()

    pltpu.sync_copy(acc.at[pl.ds(sid * RPT, RPT)],
                    out_hbm.at[cid, pl.ds(sid * RPT, RPT)])


@functools.cache
def _triple_call():
    return pl.kernel(
        _triple_body,
        out_type=jax.ShapeDtypeStruct((NW, 16), jnp.float32),
        mesh=_sc_mesh(),
        compiler_params=pltpu.CompilerParams(needs_layout_passes=False),
        scratch_types=[
            pltpu.VMEM((TPW,), jnp.int32),
            pltpu.VMEM((TPW,), jnp.int32),
            pltpu.VMEM((TPW,), jnp.int32),
            pltpu.VMEM((TCK, D), jnp.float32), pltpu.VMEM((TCK, D), jnp.float32),
            pltpu.VMEM((TCK, D), jnp.float32), pltpu.VMEM((TCK, D), jnp.float32),
            pltpu.VMEM((TCK, D), jnp.float32), pltpu.VMEM((TCK, D), jnp.float32),
            pltpu.VMEM((16,), jnp.float32),
            pltpu.SemaphoreType.DMA, pltpu.SemaphoreType.DMA,
        ],
    )


def _shuffle(v, idx):
    return lax.gather(
        v, idx[:, None],
        lax.GatherDimensionNumbers(offset_dims=(), collapsed_slice_dims=(0,),
                                   start_index_map=(0,)),
        (1,), mode=lax.GatherScatterMode.PROMISE_IN_BOUNDS)


def _triple_gathers(tab_hbm, idxs, bufs, c, sem, issue):
    for iv, dst in zip(idxs, bufs):
        src = tab_hbm.at[iv.at[pl.ds(c * TCK, TCK)]]
        if issue:
            pltpu.async_copy(src, dst, sem)
        else:
            pltpu.make_async_copy(src, dst, sem).wait()


def _triple_body(tab_hbm, a_hbm, p_hbm, n_hbm, out_hbm,
                 av, pv, nv, arA, arB, prA, prB, nrA, nrB, obuf, ts0, ts1):
    ts = (ts0, ts1)
    bufs = ((arA, prA, nrA), (arB, prB, nrB))
    idxs = (av, pv, nv)
    cid = lax.axis_index("c")
    sid = lax.axis_index("s")
    wid = sid * NC + cid
    pltpu.sync_copy(a_hbm.at[wid], av)
    pltpu.sync_copy(p_hbm.at[wid], pv)
    pltpu.sync_copy(n_hbm.at[wid], nv)

    _triple_gathers(tab_hbm, idxs, bufs[0], 0, ts[0], True)
    lanes = lax.iota(jnp.int32, 16)
    tot = jnp.float32(0.0)
    for c in range(NTCH):
        tb = c % 2
        A, P, N = bufs[tb]
        _triple_gathers(tab_hbm, idxs, bufs[tb], c, ts[tb], False)
        if c + 1 < NTCH:
            _triple_gathers(tab_hbm, idxs, bufs[1 - tb], c + 1,
                            ts[1 - tb], True)

        def tri(e, acc2):
            accp = jnp.zeros((16,), jnp.float32)
            accn = jnp.zeros((16,), jnp.float32)
            a0 = p0 = n0 = None
            for j in range(8):
                sl = pl.ds(j * 16, 16)
                a = A[e, sl]
                pp = P[e, sl]
                nn = N[e, sl]
                if j == 0:
                    a0, p0, n0 = a[0], pp[0], nn[0]
                accp = accp + a * pp
                accn = accn + a * nn
            for k in (1, 2, 4, 8):
                idx = lanes ^ k
                accp = accp + _shuffle(accp, idx)
                accn = accn + _shuffle(accn, idx)
            ip = accp[0] - 2.0 * a0 * p0
            inn = accn[0] - 2.0 * a0 * n0
            dpos = jnp.maximum(-2.0 - 2.0 * ip, 0.0)
            dneg = jnp.maximum(-2.0 - 2.0 * inn, 0.0)
            return acc2 + jnp.maximum(dpos - dneg + MARGIN, 0.0)

        tot = lax.fori_loop(0, TCK, tri, tot)

    obuf[...] = jnp.zeros((16,), jnp.float32) + tot
    pltpu.sync_copy(obuf, out_hbm.at[wid])


def kernel(edge_index, edge_weight, triples, eps, emb_user, emb_item,
           tr_gW, tr_gb, tr_eW, tr_eb, mu_W, mu_b, lv_W, lv_b,
           fc1_W, fc1_b, fc2_W, fc2_b, dec_gW, dec_gb, dec_eW, dec_eb):
    emb = jnp.concatenate([emb_user, emb_item], axis=0)
    tr_eWt = tr_eW.transpose(1, 0, 2).reshape(D, 8 * D)
    dec_eWt = dec_eW.transpose(1, 0, 2).reshape(D, 8 * D)
    m_ui = _encode_call(emb, tr_gW, tr_gb.reshape(1, 8), tr_eWt, tr_eb)

    pad = E_PAD - N_EDGES_IN
    row = jnp.pad(edge_index[0], (0, pad))
    col = jnp.pad(edge_index[1], (0, pad))
    warr = jnp.pad(edge_weight, (0, pad)).reshape(NW, NCH, CK)
    cols_w = col.reshape(NW, NCH, CK)
    rows_w = row.reshape(NW, NCH, CK)

    prev = jnp.pad(m_ui, ((0, NPAD - N_NODES), (0, 0)))
    spmm = _spmm_call()
    p = spmm(prev, cols_w, rows_w, warr)

    eps_b, p = lax.optimization_barrier((eps, p))
    recon_sum, kl_sum = _vae_call(
        m_ui, eps_b,
        mu_W, mu_b.reshape(1, D), lv_W, lv_b.reshape(1, D),
        fc1_W, fc1_b.reshape(1, 2 * D), fc2_W, fc2_b.reshape(1, D),
        dec_gW, dec_gb.reshape(1, 8), dec_eWt, dec_eb,
    )
    recon_loss = recon_sum[0, 0] / N_NODES
    kl_loss = -0.5 * kl_sum[0, 0] / (N_NODES * D)

    out1 = _add2_call(p)
    q = spmm(out1, cols_w, rows_w, warr)
    table = _finish_call(out1, q)

    a_w = triples[:, 0].reshape(NW, TPW)
    p_w = triples[:, 1].reshape(NW, TPW)
    n_w = triples[:, 2].reshape(NW, TPW)
    hinge = _triple_call()(table, a_w, p_w, n_w)
    margin_loss = jnp.sum(hinge[:, 0])

    return margin_loss + VAE_BETA * (recon_loss + kl_loss)


# speedup vs baseline: 1.3474x; 1.1393x over previous
import functools

import jax
import jax.numpy as jnp
from jax import lax
from jax.experimental import pallas as pl
from jax.experimental.pallas import tpu as pltpu
from jax.experimental.pallas import tpu_sc as plsc

D = 128
N_NODES = 10000
NPAD = 10240
RPT = NPAD // 16
NC, NS = 2, 16
NW = NC * NS
N_EDGES_IN = 320000
E_PAD = 327680
EPW = E_PAD // NW
CK = 128
NCH = EPW // CK
NT = 16384
TPW = NT // NW
TCK = 128
NTCH = TPW // TCK
MARGIN = 0.1
VAE_BETA = 0.2
MAX_NORM = 1.5
BN = 1000
NBLK = N_NODES // BN


def _col0(x):
    return lax.broadcasted_iota(jnp.int32, x.shape, 1) == 0


def _logmap0(x):
    t = jnp.clip(x[:, :1], 1.0 + 1e-7, None)
    dist = jnp.log(t + jnp.sqrt(t * t - 1.0))
    n2 = jnp.maximum(jnp.sum(x * x, axis=-1, keepdims=True) - x[:, :1] * x[:, :1], 0.0)
    n = jnp.clip(jnp.sqrt(n2), 1e-7, None)
    return jnp.where(_col0(x), 0.0, x * (dist / n))


def _expmap0(u, project):
    n2 = jnp.maximum(jnp.sum(u * u, axis=-1, keepdims=True) - u[:, :1] * u[:, :1], 0.0)
    n = jnp.clip(jnp.sqrt(n2), 1e-7, None)
    nc = jnp.minimum(n, MAX_NORM) if project else n
    e = jnp.exp(nc)
    ei = 1.0 / e
    cosh = 0.5 * (e + ei)
    sinh = 0.5 * (e - ei)
    return jnp.where(_col0(u), cosh, u * (sinh / n))


def _sqdist(x, y):
    ip = jnp.sum(x * y, axis=-1) - 2.0 * x[:, 0] * y[:, 0]
    return jnp.maximum(-2.0 - 2.0 * ip, 0.0)


def _moe_block(x, gW, gb, eWt, eb):
    g = jnp.dot(x, gW, preferred_element_type=jnp.float32) + gb
    g = g - jnp.max(g, axis=-1, keepdims=True)
    w = jnp.exp(g)
    w = w / jnp.sum(w, axis=-1, keepdims=True)
    acc = jnp.zeros((x.shape[0], D), jnp.float32)
    for ex in range(8):
        ye = jnp.dot(x, eWt[:, ex * D:(ex + 1) * D], preferred_element_type=jnp.float32)
        acc = acc + w[:, ex:ex + 1] * (ye + eb[ex:ex + 1, :])
    return acc


def _dense_body(emb_ref, eps_ref, tr_gW, tr_gb, tr_eWt, tr_eb,
                mu_W, mu_b, lv_W, lv_b, fc1_W, fc1_b, fc2_W, fc2_b,
                dec_gW, dec_gb, dec_eWt, dec_eb,
                mui_ref, recon_ref, kl_ref):
    x = emb_ref[...]
    t = _logmap0(x)
    m = _moe_block(t, tr_gW[...], tr_gb[...], tr_eWt[...], tr_eb[...])
    mui_ref[...] = m
    mu = jnp.dot(m, mu_W[...], preferred_element_type=jnp.float32) + mu_b[...]
    lv = jnp.dot(m, lv_W[...], preferred_element_type=jnp.float32) + lv_b[...]
    std = jnp.exp(0.5 * lv)
    z = _expmap0(mu + eps_ref[...] * std, True)
    zt = _logmap0(z)
    h = jnp.maximum(jnp.dot(zt, fc1_W[...], preferred_element_type=jnp.float32) + fc1_b[...], 0.0)
    h = jnp.dot(h, fc2_W[...], preferred_element_type=jnp.float32) + fc2_b[...]
    h = _moe_block(h, dec_gW[...], dec_gb[...], dec_eWt[...], dec_eb[...])
    recon = _expmap0(h, True)
    target = _expmap0(m, False)
    rblk = jnp.sum(_sqdist(recon, target))
    kblk = jnp.sum(1.0 + lv - mu * mu - jnp.exp(lv))

    @pl.when(pl.program_id(0) == 0)
    def _init():
        recon_ref[...] = jnp.zeros_like(recon_ref)
        kl_ref[...] = jnp.zeros_like(kl_ref)

    recon_ref[...] = recon_ref[...] + rblk
    kl_ref[...] = kl_ref[...] + kblk


def _full_spec(shape):
    return pl.BlockSpec(shape, lambda i: tuple(0 for _ in shape))


_dense_call = pl.pallas_call(
    _dense_body,
    grid=(NBLK,),
    in_specs=[
        pl.BlockSpec((BN, D), lambda i: (i, 0)),
        pl.BlockSpec((BN, D), lambda i: (i, 0)),
        _full_spec((D, 8)), _full_spec((1, 8)), _full_spec((D, 8 * D)), _full_spec((8, D)),
        _full_spec((D, D)), _full_spec((1, D)), _full_spec((D, D)), _full_spec((1, D)),
        _full_spec((D, 2 * D)), _full_spec((1, 2 * D)), _full_spec((2 * D, D)), _full_spec((1, D)),
        _full_spec((D, 8)), _full_spec((1, 8)), _full_spec((D, 8 * D)), _full_spec((8, D)),
    ],
    out_specs=[
        pl.BlockSpec((BN, D), lambda i: (i, 0)),
        pl.BlockSpec((1, 1), lambda i: (0, 0)),
        pl.BlockSpec((1, 1), lambda i: (0, 0)),
    ],
    out_shape=[
        jax.ShapeDtypeStruct((N_NODES, D), jnp.float32),
        jax.ShapeDtypeStruct((1, 1), jnp.float32),
        jax.ShapeDtypeStruct((1, 1), jnp.float32),
    ],
)


_CB = 1280


def _add2_body(p_ref, o_ref):
    o_ref[...] = p_ref[0] + p_ref[1]


_add2_call = pl.pallas_call(
    _add2_body,
    grid=(NPAD // _CB,),
    in_specs=[pl.BlockSpec((2, _CB, D), lambda i: (0, i, 0))],
    out_specs=pl.BlockSpec((_CB, D), lambda i: (i, 0)),
    out_shape=jax.ShapeDtypeStruct((NPAD, D), jnp.float32),
)


def _finish_body(out1_ref, q_ref, tab_ref):
    s = out1_ref[...] + q_ref[0] + q_ref[1]
    tab_ref[...] = _expmap0(s, True)


_finish_call = pl.pallas_call(
    _finish_body,
    grid=(NPAD // _CB,),
    in_specs=[pl.BlockSpec((_CB, D), lambda i: (i, 0)),
              pl.BlockSpec((2, _CB, D), lambda i: (0, i, 0))],
    out_specs=pl.BlockSpec((_CB, D), lambda i: (i, 0)),
    out_shape=jax.ShapeDtypeStruct((NPAD, D), jnp.float32),
)


def _sc_mesh():
    return plsc.VectorSubcoreMesh(core_axis_name="c", subcore_axis_name="s",
                                  num_cores=NC, num_subcores=NS)


@functools.cache
def _spmm_call():
    return pl.kernel(
        _spmm_body,
        out_type=jax.ShapeDtypeStruct((NC, NPAD, D), jnp.float32),
        mesh=_sc_mesh(),
        compiler_params=pltpu.CompilerParams(needs_layout_passes=False),
        scratch_types=[
            pltpu.VMEM((2, CK), jnp.int32), pltpu.VMEM((2, CK), jnp.int32),
            pltpu.VMEM((2, CK), jnp.int32), pltpu.VMEM((2, CK), jnp.int32),
            pltpu.VMEM((CK,), jnp.float32), pltpu.VMEM((CK,), jnp.float32),
            pltpu.VMEM((CK,), jnp.float32), pltpu.VMEM((CK,), jnp.float32),
            pltpu.VMEM((CK, D), jnp.float32), pltpu.VMEM((CK, D), jnp.float32),
            pltpu.VMEM_SHARED((NPAD, D), jnp.float32),
            pltpu.SemaphoreType.DMA, pltpu.SemaphoreType.DMA,
            pltpu.SemaphoreType.DMA, pltpu.SemaphoreType.DMA,
            pltpu.SemaphoreType.DMA, pltpu.SemaphoreType.DMA,
            pltpu.SemaphoreType.DMA, pltpu.SemaphoreType.DMA,
        ],
    )


def _spmm_body(prev_hbm, cols_hbm, rows_hbm, w_hbm, out_hbm,
               e0, e1, e2, e3, w0, w1, w2, w3, gathA, gathB, acc,
               es0, es1, es2, es3, gs0, gs1, ss0, ss1):
    eb = (e0, e1, e2, e3)
    wb = (w0, w1, w2, w3)
    gth = (gathA, gathB)
    es = (es0, es1, es2, es3)
    gs = (gs0, gs1)
    ss = (ss0, ss1)
    cid = lax.axis_index("c")
    sid = lax.axis_index("s")
    wid = sid * NC + cid
    zero = jnp.zeros((16,), jnp.float32)

    def zrow(r, carry):
        for j in range(8):
            gathA[r, pl.ds(j * 16, 16)] = zero
        return carry

    lax.fori_loop(0, CK, zrow, 0)
    for k in range(RPT // CK):
        pltpu.sync_copy(gathA, acc.at[pl.ds(sid * RPT + k * CK, CK)])
    plsc.subcore_barrier()

    def fetch_edata(c, k, issue):
        pair = ((cols_hbm.at[wid, c], eb[k].at[0]),
                (rows_hbm.at[wid, c], eb[k].at[1]),
                (w_hbm.at[wid, c], wb[k]))
        for src, dst in pair:
            if issue:
                pltpu.async_copy(src, dst, es[k])
            else:
                pltpu.make_async_copy(src, dst, es[k]).wait()

    for e in range(3):
        fetch_edata(e, e, True)
    fetch_edata(0, 0, False)
    pltpu.async_copy(prev_hbm.at[eb[0].at[0]], gathA, gs[0])

    def outer(g, carry):
        for b in range(4):
            c = g * 4 + b
            gb = b % 2
            ogb = 1 - gb
            pltpu.make_async_copy(prev_hbm.at[eb[b].at[0]],
                                  gth[gb], gs[gb]).wait()

            def grp(g16, cc):
                wvec = wb[b][pl.ds(g16 * 16, 16)]
                for l in range(16):
                    e2_ = g16 * 16 + l
                    wgt = wvec[l]
                    for j in range(8):
                        sl = pl.ds(j * 16, 16)
                        gth[gb][e2_, sl] = gth[gb][e2_, sl] * wgt
                return cc

            lax.fori_loop(0, CK // 16, grp, 0)
            pltpu.async_copy(gth[gb], acc.at[eb[b].at[1]], ss[gb], add=True)

            @pl.when(c + 1 < NCH)
            def _():
                @pl.when(c >= 1)
                def _():
                    pltpu.make_async_copy(
                        gth[ogb], acc.at[eb[(b + 3) % 4].at[1]],
                        ss[ogb]).wait()
                fetch_edata(0, (b + 1) % 4, False)
                pltpu.async_copy(prev_hbm.at[eb[(b + 1) % 4].at[0]],
                                 gth[ogb], gs[ogb])

            @pl.when(c + 3 < NCH)
            def _():
                fetch_edata(c + 3, (b + 3) % 4, True)
        return carry

    lax.fori_loop(0, NCH // 4, outer, 0)
    pltpu.make_async_copy(gathB, acc.at[eb[3].at[1]], ss[1]).wait()
    plsc.subcore_barrier()

    pltpu.sync_copy(acc.at[pl.ds(sid * RPT, RPT)],
                    out_hbm.at[cid, pl.ds(sid * RPT, RPT)])


@functools.cache
def _triple_call():
    return pl.kernel(
        _triple_body,
        out_type=jax.ShapeDtypeStruct((NW, 16), jnp.float32),
        mesh=_sc_mesh(),
        compiler_params=pltpu.CompilerParams(needs_layout_passes=False),
        scratch_types=[
            pltpu.VMEM((TPW,), jnp.int32),
            pltpu.VMEM((TPW,), jnp.int32),
            pltpu.VMEM((TPW,), jnp.int32),
            pltpu.VMEM((TCK, D), jnp.float32), pltpu.VMEM((TCK, D), jnp.float32),
            pltpu.VMEM((TCK, D), jnp.float32), pltpu.VMEM((TCK, D), jnp.float32),
            pltpu.VMEM((TCK, D), jnp.float32), pltpu.VMEM((TCK, D), jnp.float32),
            pltpu.VMEM((16,), jnp.float32),
            pltpu.SemaphoreType.DMA, pltpu.SemaphoreType.DMA,
        ],
    )


def _shuffle(v, idx):
    return lax.gather(
        v, idx[:, None],
        lax.GatherDimensionNumbers(offset_dims=(), collapsed_slice_dims=(0,),
                                   start_index_map=(0,)),
        (1,), mode=lax.GatherScatterMode.PROMISE_IN_BOUNDS)


def _triple_gathers(tab_hbm, idxs, bufs, c, sem, issue):
    for iv, dst in zip(idxs, bufs):
        src = tab_hbm.at[iv.at[pl.ds(c * TCK, TCK)]]
        if issue:
            pltpu.async_copy(src, dst, sem)
        else:
            pltpu.make_async_copy(src, dst, sem).wait()


def _triple_body(tab_hbm, a_hbm, p_hbm, n_hbm, out_hbm,
                 av, pv, nv, arA, arB, prA, prB, nrA, nrB, obuf, ts0, ts1):
    ts = (ts0, ts1)
    bufs = ((arA, prA, nrA), (arB, prB, nrB))
    idxs = (av, pv, nv)
    cid = lax.axis_index("c")
    sid = lax.axis_index("s")
    wid = sid * NC + cid
    pltpu.sync_copy(a_hbm.at[wid], av)
    pltpu.sync_copy(p_hbm.at[wid], pv)
    pltpu.sync_copy(n_hbm.at[wid], nv)

    _triple_gathers(tab_hbm, idxs, bufs[0], 0, ts[0], True)
    lanes = lax.iota(jnp.int32, 16)
    tot = jnp.float32(0.0)
    for c in range(NTCH):
        tb = c % 2
        A, P, N = bufs[tb]
        _triple_gathers(tab_hbm, idxs, bufs[tb], c, ts[tb], False)
        if c + 1 < NTCH:
            _triple_gathers(tab_hbm, idxs, bufs[1 - tb], c + 1,
                            ts[1 - tb], True)

        def tri(e, acc2):
            accp = jnp.zeros((16,), jnp.float32)
            accn = jnp.zeros((16,), jnp.float32)
            a0 = p0 = n0 = None
            for j in range(8):
                sl = pl.ds(j * 16, 16)
                a = A[e, sl]
                pp = P[e, sl]
                nn = N[e, sl]
                if j == 0:
                    a0, p0, n0 = a[0], pp[0], nn[0]
                accp = accp + a * pp
                accn = accn + a * nn
            for k in (1, 2, 4, 8):
                idx = lanes ^ k
                accp = accp + _shuffle(accp, idx)
                accn = accn + _shuffle(accn, idx)
            ip = accp[0] - 2.0 * a0 * p0
            inn = accn[0] - 2.0 * a0 * n0
            dpos = jnp.maximum(-2.0 - 2.0 * ip, 0.0)
            dneg = jnp.maximum(-2.0 - 2.0 * inn, 0.0)
            return acc2 + jnp.maximum(dpos - dneg + MARGIN, 0.0)

        tot = lax.fori_loop(0, TCK, tri, tot)

    obuf[...] = jnp.zeros((16,), jnp.float32) + tot
    pltpu.sync_copy(obuf, out_hbm.at[wid])


def kernel(edge_index, edge_weight, triples, eps, emb_user, emb_item,
           tr_gW, tr_gb, tr_eW, tr_eb, mu_W, mu_b, lv_W, lv_b,
           fc1_W, fc1_b, fc2_W, fc2_b, dec_gW, dec_gb, dec_eW, dec_eb):
    emb = jnp.concatenate([emb_user, emb_item], axis=0)
    tr_eWt = tr_eW.transpose(1, 0, 2).reshape(D, 8 * D)
    dec_eWt = dec_eW.transpose(1, 0, 2).reshape(D, 8 * D)
    m_ui, recon_sum, kl_sum = _dense_call(
        emb, eps,
        tr_gW, tr_gb.reshape(1, 8), tr_eWt, tr_eb,
        mu_W, mu_b.reshape(1, D), lv_W, lv_b.reshape(1, D),
        fc1_W, fc1_b.reshape(1, 2 * D), fc2_W, fc2_b.reshape(1, D),
        dec_gW, dec_gb.reshape(1, 8), dec_eWt, dec_eb,
    )
    recon_loss = recon_sum[0, 0] / N_NODES
    kl_loss = -0.5 * kl_sum[0, 0] / (N_NODES * D)

    pad = E_PAD - N_EDGES_IN
    row = jnp.pad(edge_index[0], (0, pad))
    col = jnp.pad(edge_index[1], (0, pad))
    warr = jnp.pad(edge_weight, (0, pad)).reshape(NW, NCH, CK)
    cols_w = col.reshape(NW, NCH, CK)
    rows_w = row.reshape(NW, NCH, CK)

    prev = jnp.pad(m_ui, ((0, NPAD - N_NODES), (0, 0)))
    spmm = _spmm_call()
    p = spmm(prev, cols_w, rows_w, warr)
    out1 = _add2_call(p)
    q = spmm(out1, cols_w, rows_w, warr)
    table = _finish_call(out1, q)

    a_w = triples[:, 0].reshape(NW, TPW)
    p_w = triples[:, 1].reshape(NW, TPW)
    n_w = triples[:, 2].reshape(NW, TPW)
    hinge = _triple_call()(table, a_w, p_w, n_w)
    margin_loss = jnp.sum(hinge[:, 0])

    return margin_loss + VAE_BETA * (recon_loss + kl_loss)
